# baseline (speedup 1.0000x reference)
import functools

import jax
import jax.numpy as jnp
from jax import lax
from jax.experimental import pallas as pl
from jax.experimental.pallas import tpu as pltpu
from jax.experimental.pallas import tpu_sc as plsc

NUM_EMB = 100000
D = 128
ROW_BLOCK = 25000

NC = 2
NS = 16
NW = NC * NS
CHUNK = 128


def _transform_body(x_ref, w_ref, b_ref, o_ref):
    o_ref[...] = (
        lax.dot_general(
            x_ref[...], w_ref[...],
            (((1,), (1,)), ((), ())),
            preferred_element_type=jnp.float32,
        )
        + b_ref[...]
    )


def _transform_table(table, W, b):
    return pl.pallas_call(
        _transform_body,
        grid=(NUM_EMB // ROW_BLOCK,),
        in_specs=[
            pl.BlockSpec((ROW_BLOCK, D), lambda i: (i, 0)),
            pl.BlockSpec((D, D), lambda i: (0, 0)),
            pl.BlockSpec((1, D), lambda i: (0, 0)),
        ],
        out_specs=pl.BlockSpec((ROW_BLOCK, D), lambda i: (i, 0)),
        out_shape=jax.ShapeDtypeStruct((NUM_EMB, D), jnp.float32),
    )(table, W, b.reshape(1, D))


NSLOT = 5


def _make_gather(b_flat):
    b_per_w = b_flat // NW
    n_chunks = b_per_w // CHUNK
    n_outer = n_chunks // NSLOT

    @functools.partial(
        pl.kernel,
        mesh=plsc.VectorSubcoreMesh(core_axis_name="c", subcore_axis_name="s"),
        out_type=jax.ShapeDtypeStruct((b_flat, D), jnp.float32),
        scratch_types=(
            [
                pltpu.VMEM((b_per_w,), jnp.int32),
                pltpu.VMEM((NSLOT, CHUNK, D), jnp.float32),
            ]
            + [pltpu.SemaphoreType.DMA] * (2 * NSLOT)
        ),
    )
    def gather_kernel(ids_hbm, table_hbm, out_hbm, idx_v, rows_v, *sems):
        sem_g = sems[:NSLOT]
        sem_o = sems[NSLOT:]
        wid = lax.axis_index("s") * NC + lax.axis_index("c")
        base = wid * b_per_w
        pltpu.sync_copy(ids_hbm.at[pl.ds(base, b_per_w)], idx_v)

        def gather_cp(c, s):
            return pltpu.make_async_copy(
                table_hbm.at[idx_v.at[pl.ds(c * CHUNK, CHUNK)]],
                rows_v.at[s],
                sem_g[s],
            )

        def out_cp(c, s):
            return pltpu.make_async_copy(
                rows_v.at[s],
                out_hbm.at[pl.ds(base + c * CHUNK, CHUNK)],
                sem_o[s],
            )

        for s in range(NSLOT):
            gather_cp(s, s).start()

        def body(g, carry):
            for s in range(NSLOT):
                c = g * NSLOT + s
                gather_cp(c, s).wait()
                out_cp(c, s).start()
            for s in range(NSLOT):
                c = g * NSLOT + s
                out_cp(c, s).wait()
                gather_cp(c + NSLOT, s).start()
            return carry

        lax.fori_loop(0, n_outer - 1, body, 0)

        last = (n_outer - 1) * NSLOT
        for s in range(NSLOT):
            gather_cp(last + s, s).wait()
            out_cp(last + s, s).start()
        for s in range(NSLOT):
            out_cp(last + s, s).wait()

    return gather_kernel


def kernel(appearance_ids, table, W, b):
    batch, hist = appearance_ids.shape
    ids = appearance_ids.T.reshape(-1).astype(jnp.int32)
    t2 = _transform_table(table, W, b)
    out = _make_gather(ids.shape[0])(ids, t2)
    return out.reshape(hist, batch, D).transpose(1, 0, 2)


# speedup vs baseline: 1.0065x; 1.0065x over previous
import functools

import jax
import jax.numpy as jnp
from jax import lax
from jax.experimental import pallas as pl
from jax.experimental.pallas import tpu as pltpu
from jax.experimental.pallas import tpu_sc as plsc

NUM_EMB = 100000
D = 128
ROW_BLOCK = 20000

NC = 2
NS = 16
NW = NC * NS
CHUNK = 128


def _transform_body(x_ref, w_ref, b_ref, o_ref):
    o_ref[...] = (
        lax.dot_general(
            x_ref[...], w_ref[...],
            (((1,), (1,)), ((), ())),
            preferred_element_type=jnp.float32,
        )
        + b_ref[...]
    )


def _transform_table(table, W, b):
    return pl.pallas_call(
        _transform_body,
        grid=(NUM_EMB // ROW_BLOCK,),
        in_specs=[
            pl.BlockSpec((ROW_BLOCK, D), lambda i: (i, 0)),
            pl.BlockSpec((D, D), lambda i: (0, 0)),
            pl.BlockSpec((1, D), lambda i: (0, 0)),
        ],
        out_specs=pl.BlockSpec((ROW_BLOCK, D), lambda i: (i, 0)),
        out_shape=jax.ShapeDtypeStruct((NUM_EMB, D), jnp.float32),
    )(table, W, b.reshape(1, D))


NSLOT = 5


def _make_gather(b_flat):
    b_per_w = b_flat // NW
    n_chunks = b_per_w // CHUNK
    n_outer = n_chunks // NSLOT

    @functools.partial(
        pl.kernel,
        mesh=plsc.VectorSubcoreMesh(core_axis_name="c", subcore_axis_name="s"),
        out_type=jax.ShapeDtypeStruct((b_flat, D), jnp.float32),
        scratch_types=(
            [
                pltpu.VMEM((b_per_w,), jnp.int32),
                pltpu.VMEM((NSLOT, CHUNK, D), jnp.float32),
            ]
            + [pltpu.SemaphoreType.DMA] * (2 * NSLOT)
        ),
    )
    def gather_kernel(ids_hbm, table_hbm, out_hbm, idx_v, rows_v, *sems):
        sem_g = sems[:NSLOT]
        sem_o = sems[NSLOT:]
        wid = lax.axis_index("s") * NC + lax.axis_index("c")
        base = wid * b_per_w
        pltpu.sync_copy(ids_hbm.at[pl.ds(base, b_per_w)], idx_v)

        def gather_cp(c, s):
            return pltpu.make_async_copy(
                table_hbm.at[idx_v.at[pl.ds(c * CHUNK, CHUNK)]],
                rows_v.at[s],
                sem_g[s],
            )

        def out_cp(c, s):
            return pltpu.make_async_copy(
                rows_v.at[s],
                out_hbm.at[pl.ds(base + c * CHUNK, CHUNK)],
                sem_o[s],
            )

        for s in range(NSLOT):
            gather_cp(s, s).start()

        def body(g, carry):
            for s in range(NSLOT):
                c = g * NSLOT + s
                gather_cp(c, s).wait()
                out_cp(c, s).start()
            for s in range(NSLOT):
                c = g * NSLOT + s
                out_cp(c, s).wait()
                gather_cp(c + NSLOT, s).start()
            return carry

        lax.fori_loop(0, n_outer - 1, body, 0)

        last = (n_outer - 1) * NSLOT
        for s in range(NSLOT):
            gather_cp(last + s, s).wait()
            out_cp(last + s, s).start()
        for s in range(NSLOT):
            out_cp(last + s, s).wait()

    return gather_kernel


def kernel(appearance_ids, table, W, b):
    batch, hist = appearance_ids.shape
    ids = appearance_ids.T.reshape(-1).astype(jnp.int32)
    t2 = _transform_table(table, W, b)
    out = _make_gather(ids.shape[0])(ids, t2)
    return out.reshape(hist, batch, D).transpose(1, 0, 2)
